# baseline (speedup 1.0000x reference)
import functools

import jax
import jax.numpy as jnp
from jax import lax
from jax.experimental import pallas as pl
from jax.experimental.pallas import tpu as pltpu
from jax.experimental.pallas import tpu_sc as plsc

HEADS = 16
BUCKETS = 64
DIM = 64
SEQ = 8192
RPB = SEQ // BUCKETS

BUCKETS_PER_CHUNK = 2
ROWS_PER_CHUNK = BUCKETS_PER_CHUNK * RPB
NCHUNKS = BUCKETS // BUCKETS_PER_CHUNK


def _sc_body(q_hbm, k_hbm, sq_hbm, sk_hbm, buf0, buf1, res, sem0, sem1):
    wid = lax.axis_index("s") * 2 + lax.axis_index("c")

    def accumulate_chunk(buf, c):
        def bucket(b):
            def step(s, accs):
                row = b * RPB + s
                return tuple(
                    accs[g] + buf[row, g * 16:(g + 1) * 16] for g in range(4)
                )

            accs = lax.fori_loop(
                0, RPB, step, tuple(jnp.zeros((16,), jnp.float32) for _ in range(4))
            )
            for g in range(4):
                res[c * BUCKETS_PER_CHUNK + b, g * 16:(g + 1) * 16] = accs[g]

        for b in range(BUCKETS_PER_CHUNK):
            bucket(b)

    def one_tensor(src_hbm, dst_hbm):
        def chunk_slice(c):
            return src_hbm.at[wid, pl.ds(c * ROWS_PER_CHUNK, ROWS_PER_CHUNK), :]

        cp0 = pltpu.async_copy(chunk_slice(0), buf0, sem0)
        cp0.wait()
        for c in range(NCHUNKS):
            cur, nxt = (buf0, buf1) if c % 2 == 0 else (buf1, buf0)
            nsem = sem1 if c % 2 == 0 else sem0
            if c + 1 < NCHUNKS:
                nxt_cp = pltpu.async_copy(chunk_slice(c + 1), nxt, nsem)
            accumulate_chunk(cur, c)
            if c + 1 < NCHUNKS:
                nxt_cp.wait()
        pltpu.sync_copy(res, dst_hbm.at[wid])

    one_tensor(q_hbm, sq_hbm)
    one_tensor(k_hbm, sk_hbm)


def _tc_body(sq_ref, sk_ref, pq_ref, pk_ref, out_ref):
    inv = 1.0 / RPB
    pq = jnp.concatenate((pq_ref[...], pq_ref[...]), axis=0)
    pk = jnp.concatenate((pk_ref[...], pk_ref[...]), axis=0)
    a = sq_ref[...] * inv + pq
    b = sk_ref[...] * inv + pk
    r = jax.lax.dot_general(
        a, b, (((2,), (2,)), ((0,), (0,))), preferred_element_type=jnp.float32
    )
    r = r - jnp.max(r, axis=-1, keepdims=True)
    e = jnp.exp(r)
    out_ref[...] = e / jnp.sum(e, axis=-1, keepdims=True)


def kernel(q, k, q_pos_emb, k_pos_emb):
    bh = q.shape[0]
    mesh = plsc.VectorSubcoreMesh(core_axis_name="c", subcore_axis_name="s")

    sc = pl.kernel(
        _sc_body,
        mesh=mesh,
        out_type=(
            jax.ShapeDtypeStruct((bh, BUCKETS, DIM), jnp.float32),
            jax.ShapeDtypeStruct((bh, BUCKETS, DIM), jnp.float32),
        ),
        scratch_types=[
            pltpu.VMEM((ROWS_PER_CHUNK, DIM), jnp.float32),
            pltpu.VMEM((ROWS_PER_CHUNK, DIM), jnp.float32),
            pltpu.VMEM((BUCKETS, DIM), jnp.float32),
            pltpu.SemaphoreType.DMA,
            pltpu.SemaphoreType.DMA,
        ],
    )
    sq, sk = sc(q, k)

    return pl.pallas_call(
        _tc_body,
        grid=(1,),
        in_specs=[
            pl.BlockSpec((bh, BUCKETS, DIM), lambda i: (0, 0, 0)),
            pl.BlockSpec((bh, BUCKETS, DIM), lambda i: (0, 0, 0)),
            pl.BlockSpec((HEADS, BUCKETS, DIM), lambda i: (0, 0, 0)),
            pl.BlockSpec((HEADS, BUCKETS, DIM), lambda i: (0, 0, 0)),
        ],
        out_specs=pl.BlockSpec((bh, BUCKETS, BUCKETS), lambda i: (0, 0, 0)),
        out_shape=jax.ShapeDtypeStruct((bh, BUCKETS, BUCKETS), jnp.float32),
    )(sq, sk, q_pos_emb[0], k_pos_emb[0])


# speedup vs baseline: 1.0084x; 1.0084x over previous
import functools

import jax
import jax.numpy as jnp
from jax import lax
from jax.experimental import pallas as pl
from jax.experimental.pallas import tpu as pltpu
from jax.experimental.pallas import tpu_sc as plsc

HEADS = 16
BUCKETS = 64
DIM = 64
SEQ = 8192
RPB = SEQ // BUCKETS

BUCKETS_PER_CHUNK = 2
ROWS_PER_CHUNK = BUCKETS_PER_CHUNK * RPB
NCHUNKS = BUCKETS // BUCKETS_PER_CHUNK


def _sc_body(q_hbm, k_hbm, sq_hbm, sk_hbm, buf0, buf1, res, sem0, sem1):
    wid = lax.axis_index("s") * 2 + lax.axis_index("c")

    UNROLL = 8

    def accumulate_chunk(buf, c):
        def bucket(b):
            def step(i, accs):
                row = b * RPB + i * UNROLL
                out = list(accs)
                for u in range(UNROLL):
                    for g in range(4):
                        out[g] = out[g] + buf[row + u, g * 16:(g + 1) * 16]
                return tuple(out)

            accs = lax.fori_loop(
                0, RPB // UNROLL, step,
                tuple(jnp.zeros((16,), jnp.float32) for _ in range(4)),
            )
            for g in range(4):
                res[c * BUCKETS_PER_CHUNK + b, g * 16:(g + 1) * 16] = accs[g]

        for b in range(BUCKETS_PER_CHUNK):
            bucket(b)

    def one_tensor(src_hbm, dst_hbm):
        def chunk_slice(c):
            return src_hbm.at[wid, pl.ds(c * ROWS_PER_CHUNK, ROWS_PER_CHUNK), :]

        pltpu.async_copy(chunk_slice(0), buf0, sem0)
        pltpu.async_copy(chunk_slice(1), buf1, sem1)

        def iter_j(j, carry):
            for half, (buf, sem) in enumerate(((buf0, sem0), (buf1, sem1))):
                c = 2 * j + half
                pltpu.make_async_copy(chunk_slice(0), buf, sem).wait()
                accumulate_chunk(buf, c)
                nc = jnp.minimum(c + 2, NCHUNKS - 1)
                pltpu.async_copy(chunk_slice(nc), buf, sem)
            return carry

        lax.fori_loop(0, NCHUNKS // 2, iter_j, 0)
        pltpu.make_async_copy(chunk_slice(0), buf0, sem0).wait()
        pltpu.make_async_copy(chunk_slice(0), buf1, sem1).wait()
        pltpu.sync_copy(res, dst_hbm.at[wid])

    one_tensor(q_hbm, sq_hbm)
    one_tensor(k_hbm, sk_hbm)


def _tc_body(sq_ref, sk_ref, pq_ref, pk_ref, out_ref):
    inv = 1.0 / RPB
    pq = jnp.concatenate((pq_ref[...], pq_ref[...]), axis=0)
    pk = jnp.concatenate((pk_ref[...], pk_ref[...]), axis=0)
    a = sq_ref[...] * inv + pq
    b = sk_ref[...] * inv + pk
    r = jax.lax.dot_general(
        a, b, (((2,), (2,)), ((0,), (0,))), preferred_element_type=jnp.float32
    )
    r = r - jnp.max(r, axis=-1, keepdims=True)
    e = jnp.exp(r)
    out_ref[...] = e / jnp.sum(e, axis=-1, keepdims=True)


def kernel(q, k, q_pos_emb, k_pos_emb):
    bh = q.shape[0]
    mesh = plsc.VectorSubcoreMesh(core_axis_name="c", subcore_axis_name="s")

    sc = pl.kernel(
        _sc_body,
        mesh=mesh,
        out_type=(
            jax.ShapeDtypeStruct((bh, BUCKETS, DIM), jnp.float32),
            jax.ShapeDtypeStruct((bh, BUCKETS, DIM), jnp.float32),
        ),
        scratch_types=[
            pltpu.VMEM((ROWS_PER_CHUNK, DIM), jnp.float32),
            pltpu.VMEM((ROWS_PER_CHUNK, DIM), jnp.float32),
            pltpu.VMEM((BUCKETS, DIM), jnp.float32),
            pltpu.SemaphoreType.DMA,
            pltpu.SemaphoreType.DMA,
        ],
    )
    sq, sk = sc(q, k)

    return pl.pallas_call(
        _tc_body,
        grid=(1,),
        in_specs=[
            pl.BlockSpec((bh, BUCKETS, DIM), lambda i: (0, 0, 0)),
            pl.BlockSpec((bh, BUCKETS, DIM), lambda i: (0, 0, 0)),
            pl.BlockSpec((HEADS, BUCKETS, DIM), lambda i: (0, 0, 0)),
            pl.BlockSpec((HEADS, BUCKETS, DIM), lambda i: (0, 0, 0)),
        ],
        out_specs=pl.BlockSpec((bh, BUCKETS, BUCKETS), lambda i: (0, 0, 0)),
        out_shape=jax.ShapeDtypeStruct((bh, BUCKETS, BUCKETS), jnp.float32),
    )(sq, sk, q_pos_emb[0], k_pos_emb[0])


# speedup vs baseline: 2.7413x; 2.7185x over previous
import jax
import jax.numpy as jnp
from jax import lax
from jax.experimental import pallas as pl
from jax.experimental.pallas import tpu as pltpu
from jax.experimental.pallas import tpu_sc as plsc

HEADS = 16
BUCKETS = 64
DIM = 64
SEQ = 8192
RPB = SEQ // BUCKETS
NL = 16

TC_BH = 24
SC_BH = 8
WPB = 32 // SC_BH
BPW = BUCKETS // WPB
GROUPS = 2
BPG = BPW // GROUPS


def _sc_body(q_hbm, k_hbm, sq_hbm, sk_hbm, buf0, buf1, res, sem0, sem1):
    wid = lax.axis_index("s") * 2 + lax.axis_index("c")
    bh_l = wid // WPB
    bbase = (wid % WPB) * BPW

    def accumulate_bucket(buf, local_b):
        def dim_step(d, carry):
            p = buf[d, 0:NL]
            for j in range(1, RPB // NL):
                p = p + buf[d, j * NL:(j + 1) * NL]
            res[local_b, d, :] = p
            return carry

        lax.fori_loop(0, DIM, dim_step, 0)

    def one_tensor(src_hbm, dst_hbm):
        def chunk_slice(c):
            return src_hbm.at[
                TC_BH + bh_l, :, pl.ds((bbase + c) * RPB, RPB)
            ]

        pltpu.async_copy(chunk_slice(0), buf0, sem0)
        pltpu.async_copy(chunk_slice(1), buf1, sem1)

        for grp in range(GROUPS):
            def iter_j(j, carry):
                for half, (buf, sem) in enumerate(((buf0, sem0), (buf1, sem1))):
                    c = grp * BPG + 2 * j + half
                    pltpu.make_async_copy(chunk_slice(0), buf, sem).wait()
                    accumulate_bucket(buf, 2 * j + half)
                    nc = jnp.minimum(c + 2, BPW - 1)
                    pltpu.async_copy(chunk_slice(nc), buf, sem)
                return carry

            lax.fori_loop(0, BPG // 2, iter_j, 0)
            pltpu.sync_copy(res, dst_hbm.at[bh_l, pl.ds(bbase + grp * BPG, BPG)])

        pltpu.make_async_copy(chunk_slice(0), buf0, sem0).wait()
        pltpu.make_async_copy(chunk_slice(0), buf1, sem1).wait()

    one_tensor(q_hbm, sq_hbm)
    one_tensor(k_hbm, sk_hbm)


def _tc_main_body(q_ref, k_ref, w_ref, pq_ref, pk_ref, out_ref):
    w = w_ref[...]
    a = jax.lax.dot_general(
        q_ref[0], w, (((1,), (0,)), ((), ())), preferred_element_type=jnp.float32
    ) + pq_ref[0]
    b = jax.lax.dot_general(
        k_ref[0], w, (((1,), (0,)), ((), ())), preferred_element_type=jnp.float32
    ) + pk_ref[0]
    r = jax.lax.dot_general(
        a, b, (((0,), (0,)), ((), ())), preferred_element_type=jnp.float32
    )
    r = r - jnp.max(r, axis=-1, keepdims=True)
    e = jnp.exp(r)
    out_ref[0] = e / jnp.sum(e, axis=-1, keepdims=True)


def _tc_fold_body(pq4_ref, pk4_ref, f_ref, pq_ref, pk_ref, out_ref):
    f = f_ref[...]
    a = jax.lax.dot_general(
        pq4_ref[0], f, (((1,), (0,)), ((), ())), preferred_element_type=jnp.float32
    ) + pq_ref[0]
    b = jax.lax.dot_general(
        pk4_ref[0], f, (((1,), (0,)), ((), ())), preferred_element_type=jnp.float32
    ) + pk_ref[0]
    r = jax.lax.dot_general(
        a, b, (((1,), (1,)), ((), ())), preferred_element_type=jnp.float32
    )
    r = r - jnp.max(r, axis=-1, keepdims=True)
    e = jnp.exp(r)
    out_ref[0] = e / jnp.sum(e, axis=-1, keepdims=True)


def kernel(q, k, q_pos_emb, k_pos_emb):
    bh = q.shape[0]
    qt = jnp.swapaxes(q, 1, 2)
    kt = jnp.swapaxes(k, 1, 2)
    pqt = jnp.swapaxes(q_pos_emb[0], 1, 2)
    pkt = jnp.swapaxes(k_pos_emb[0], 1, 2)

    mesh = plsc.VectorSubcoreMesh(core_axis_name="c", subcore_axis_name="s")
    sc = pl.kernel(
        _sc_body,
        mesh=mesh,
        out_type=(
            pltpu.HBM((SC_BH, BUCKETS, DIM, NL), jnp.float32),
            pltpu.HBM((SC_BH, BUCKETS, DIM, NL), jnp.float32),
        ),
        scratch_types=[
            pltpu.VMEM((DIM, RPB), jnp.float32),
            pltpu.VMEM((DIM, RPB), jnp.float32),
            pltpu.VMEM((BPG, DIM, NL), jnp.float32),
            pltpu.SemaphoreType.DMA,
            pltpu.SemaphoreType.DMA,
        ],
    )
    pq4, pk4 = sc(qt, kt)
    pq4 = pq4.reshape(SC_BH, BUCKETS, DIM * NL)
    pk4 = pk4.reshape(SC_BH, BUCKETS, DIM * NL)

    w = jnp.repeat(jnp.eye(BUCKETS, dtype=jnp.float32) / RPB, RPB, axis=0)
    out_tc = pl.pallas_call(
        _tc_main_body,
        grid=(TC_BH,),
        in_specs=[
            pl.BlockSpec((1, DIM, SEQ), lambda i: (i, 0, 0)),
            pl.BlockSpec((1, DIM, SEQ), lambda i: (i, 0, 0)),
            pl.BlockSpec((SEQ, BUCKETS), lambda i: (0, 0)),
            pl.BlockSpec((1, DIM, BUCKETS), lambda i: (i % HEADS, 0, 0)),
            pl.BlockSpec((1, DIM, BUCKETS), lambda i: (i % HEADS, 0, 0)),
        ],
        out_specs=pl.BlockSpec((1, BUCKETS, BUCKETS), lambda i: (i, 0, 0)),
        out_shape=jax.ShapeDtypeStruct((TC_BH, BUCKETS, BUCKETS), jnp.float32),
    )(qt, kt, w, pqt, pkt)

    fold = jnp.repeat(jnp.eye(DIM, dtype=jnp.float32), NL, axis=0) / RPB
    out_sc = pl.pallas_call(
        _tc_fold_body,
        grid=(SC_BH,),
        in_specs=[
            pl.BlockSpec((1, BUCKETS, DIM * NL), lambda i: (i, 0, 0)),
            pl.BlockSpec((1, BUCKETS, DIM * NL), lambda i: (i, 0, 0)),
            pl.BlockSpec((DIM * NL, DIM), lambda i: (0, 0)),
            pl.BlockSpec((1, BUCKETS, DIM), lambda i: ((TC_BH + i) % HEADS, 0, 0)),
            pl.BlockSpec((1, BUCKETS, DIM), lambda i: ((TC_BH + i) % HEADS, 0, 0)),
        ],
        out_specs=pl.BlockSpec((1, BUCKETS, BUCKETS), lambda i: (i, 0, 0)),
        out_shape=jax.ShapeDtypeStruct((SC_BH, BUCKETS, BUCKETS), jnp.float32),
    )(pq4, pk4, fold, q_pos_emb[0], k_pos_emb[0])

    return jnp.concatenate((out_tc, out_sc), axis=0)


# speedup vs baseline: 4.0280x; 1.4694x over previous
import jax
import jax.numpy as jnp
from jax import lax
from jax.experimental import pallas as pl
from jax.experimental.pallas import tpu as pltpu
from jax.experimental.pallas import tpu_sc as plsc

HEADS = 16
BUCKETS = 64
DIM = 64
SEQ = 8192
RPB = SEQ // BUCKETS
NL = 16
PW = DIM * NL

TC_BH = 24
SC_BH = 8
WPB = 32 // SC_BH
BPW = BUCKETS // WPB
GROUPS = 2
BPG = BPW // GROUPS


def _sc_body(q_hbm, k_hbm, sq_hbm, sk_hbm, buf0, buf1, res, sem0, sem1):
    wid = lax.axis_index("s") * 2 + lax.axis_index("c")
    bh_l = wid // WPB
    bbase = (wid % WPB) * BPW

    def accumulate_bucket(buf, local_b):
        def dimsum(d):
            v = [buf[d, j * NL:(j + 1) * NL] for j in range(RPB // NL)]
            return ((v[0] + v[1]) + (v[2] + v[3])) + ((v[4] + v[5]) + (v[6] + v[7]))

        def pair_step(d2, carry):
            d = 2 * d2
            res[local_b, pl.ds(d * NL, NL)] = dimsum(d)
            res[local_b, pl.ds((d + 1) * NL, NL)] = dimsum(d + 1)
            return carry

        lax.fori_loop(0, DIM // 2, pair_step, 0)

    def one_tensor(src_hbm, dst_hbm):
        def chunk_slice(c):
            return src_hbm.at[
                TC_BH + bh_l, :, pl.ds((bbase + c) * RPB, RPB)
            ]

        pltpu.async_copy(chunk_slice(0), buf0, sem0)
        pltpu.async_copy(chunk_slice(1), buf1, sem1)

        for grp in range(GROUPS):
            def iter_j(j, carry):
                for half, (buf, sem) in enumerate(((buf0, sem0), (buf1, sem1))):
                    c = grp * BPG + 2 * j + half
                    pltpu.make_async_copy(chunk_slice(0), buf, sem).wait()
                    accumulate_bucket(buf, 2 * j + half)
                    nc = jnp.minimum(c + 2, BPW - 1)
                    pltpu.async_copy(chunk_slice(nc), buf, sem)
                return carry

            lax.fori_loop(0, BPG // 2, iter_j, 0)
            pltpu.sync_copy(res, dst_hbm.at[bh_l, pl.ds(bbase + grp * BPG, BPG)])

        pltpu.make_async_copy(chunk_slice(0), buf0, sem0).wait()
        pltpu.make_async_copy(chunk_slice(0), buf1, sem1).wait()

    one_tensor(q_hbm, sq_hbm)
    one_tensor(k_hbm, sk_hbm)


def _tc_main_body(q_ref, k_ref, w_ref, pq_ref, pk_ref, out_ref):
    w = w_ref[...]
    a = jax.lax.dot_general(
        q_ref[0], w, (((1,), (0,)), ((), ())), preferred_element_type=jnp.float32
    ) + pq_ref[0]
    b = jax.lax.dot_general(
        k_ref[0], w, (((1,), (0,)), ((), ())), preferred_element_type=jnp.float32
    ) + pk_ref[0]
    r = jax.lax.dot_general(
        a, b, (((0,), (0,)), ((), ())), preferred_element_type=jnp.float32
    )
    r = r - jnp.max(r, axis=-1, keepdims=True)
    e = jnp.exp(r)
    out_ref[0] = e / jnp.sum(e, axis=-1, keepdims=True)


def _tc_fold_body(pq4_ref, pk4_ref, f_ref, pq_ref, pk_ref, out_ref):
    f = f_ref[...]
    a = jax.lax.dot_general(
        pq4_ref[0], f, (((1,), (0,)), ((), ())), preferred_element_type=jnp.float32
    ) + pq_ref[0]
    b = jax.lax.dot_general(
        pk4_ref[0], f, (((1,), (0,)), ((), ())), preferred_element_type=jnp.float32
    ) + pk_ref[0]
    r = jax.lax.dot_general(
        a, b, (((1,), (1,)), ((), ())), preferred_element_type=jnp.float32
    )
    r = r - jnp.max(r, axis=-1, keepdims=True)
    e = jnp.exp(r)
    out_ref[0] = e / jnp.sum(e, axis=-1, keepdims=True)


def kernel(q, k, q_pos_emb, k_pos_emb):
    bh = q.shape[0]
    qt = jnp.swapaxes(q, 1, 2)
    kt = jnp.swapaxes(k, 1, 2)
    pqt = jnp.swapaxes(q_pos_emb[0], 1, 2)
    pkt = jnp.swapaxes(k_pos_emb[0], 1, 2)

    mesh = plsc.VectorSubcoreMesh(core_axis_name="c", subcore_axis_name="s")
    sc = pl.kernel(
        _sc_body,
        mesh=mesh,
        out_type=(
            pltpu.HBM((SC_BH, BUCKETS, PW), jnp.float32),
            pltpu.HBM((SC_BH, BUCKETS, PW), jnp.float32),
        ),
        scratch_types=[
            pltpu.VMEM((DIM, RPB), jnp.float32),
            pltpu.VMEM((DIM, RPB), jnp.float32),
            pltpu.VMEM((BPG, PW), jnp.float32),
            pltpu.SemaphoreType.DMA,
            pltpu.SemaphoreType.DMA,
        ],
    )
    pq4, pk4 = sc(qt, kt)

    w = jnp.repeat(jnp.eye(BUCKETS, dtype=jnp.float32) / RPB, RPB, axis=0)
    out_tc = pl.pallas_call(
        _tc_main_body,
        grid=(TC_BH,),
        in_specs=[
            pl.BlockSpec((1, DIM, SEQ), lambda i: (i, 0, 0)),
            pl.BlockSpec((1, DIM, SEQ), lambda i: (i, 0, 0)),
            pl.BlockSpec((SEQ, BUCKETS), lambda i: (0, 0)),
            pl.BlockSpec((1, DIM, BUCKETS), lambda i: (i % HEADS, 0, 0)),
            pl.BlockSpec((1, DIM, BUCKETS), lambda i: (i % HEADS, 0, 0)),
        ],
        out_specs=pl.BlockSpec((1, BUCKETS, BUCKETS), lambda i: (i, 0, 0)),
        out_shape=jax.ShapeDtypeStruct((TC_BH, BUCKETS, BUCKETS), jnp.float32),
    )(qt, kt, w, pqt, pkt)

    fidx = jnp.arange(PW) // NL
    fold = (fidx[:, None] == jnp.arange(DIM)[None, :]).astype(jnp.float32) / RPB
    out_sc = pl.pallas_call(
        _tc_fold_body,
        grid=(SC_BH,),
        in_specs=[
            pl.BlockSpec((1, BUCKETS, PW), lambda i: (i, 0, 0)),
            pl.BlockSpec((1, BUCKETS, PW), lambda i: (i, 0, 0)),
            pl.BlockSpec((PW, DIM), lambda i: (0, 0)),
            pl.BlockSpec((1, BUCKETS, DIM), lambda i: ((TC_BH + i) % HEADS, 0, 0)),
            pl.BlockSpec((1, BUCKETS, DIM), lambda i: ((TC_BH + i) % HEADS, 0, 0)),
        ],
        out_specs=pl.BlockSpec((1, BUCKETS, BUCKETS), lambda i: (i, 0, 0)),
        out_shape=jax.ShapeDtypeStruct((SC_BH, BUCKETS, BUCKETS), jnp.float32),
    )(pq4, pk4, fold, q_pos_emb[0], k_pos_emb[0])

    return jnp.concatenate((out_tc, out_sc), axis=0)
